# baseline (speedup 1.0000x reference)
import functools

import jax
import jax.numpy as jnp
from jax import lax
from jax.experimental import pallas as pl
from jax.experimental.pallas import tpu as pltpu
from jax.experimental.pallas import tpu_sc as plsc

B = 16384
K = 32
NC = 2
NS = 16
NW = NC * NS
BPW = B // NW
CHUNK = 128
NCHUNK = BPW // CHUNK


def _sc_gmf(uid_hbm, iid_hbm, wb_hbm, user_hbm, item_hbm, out_hbm,
            idx_u, idx_i, u_rows, i_rows, wb_v, out_v, sem):
    wid = lax.axis_index("s") * NC + lax.axis_index("c")
    base = wid * BPW

    pltpu.sync_copy(uid_hbm.at[pl.ds(base, BPW)], idx_u)
    pltpu.sync_copy(iid_hbm.at[pl.ds(base, BPW)], idx_i)
    pltpu.sync_copy(wb_hbm, wb_v)

    copies = []
    for c in range(NCHUNK):
        sl = pl.ds(c * CHUNK, CHUNK)
        copies.append(pltpu.async_copy(user_hbm.at[idx_u.at[sl]], u_rows.at[sl], sem))
        copies.append(pltpu.async_copy(item_hbm.at[idx_i.at[sl]], i_rows.at[sl], sem))
    for cp in copies:
        cp.wait()

    iota16 = lax.iota(jnp.int32, 16)
    w_lo = wb_v[pl.ds(0, 16)]
    w_hi = wb_v[pl.ds(16, 16)]
    b_vec = wb_v[pl.ds(K, 16)]
    bias = jnp.zeros((16,), jnp.float32) + b_vec[0]

    perm_idx = [iota16 ^ s for s in (1, 2, 4, 8)]

    def _xor_perm(v, level):
        return v.at[perm_idx[level]].get(mode="promise_in_bounds",
                                         unique_indices=True)

    def blk_body(blk, carry):
        vecs = []
        for j in range(16):
            r = blk * 16 + j
            u0 = u_rows[r, pl.ds(0, 16)]
            u1 = u_rows[r, pl.ds(16, 16)]
            i0 = i_rows[r, pl.ds(0, 16)]
            i1 = i_rows[r, pl.ds(16, 16)]
            vecs.append(u0 * i0 * w_lo + u1 * i1 * w_hi)
        for level, s in enumerate((1, 2, 4, 8)):
            lane_bit = (iota16 & s) == 0
            nxt = []
            for j in range(0, len(vecs), 2):
                a = vecs[j] + _xor_perm(vecs[j], level)
                b = vecs[j + 1] + _xor_perm(vecs[j + 1], level)
                nxt.append(jnp.where(lane_bit, a, b))
            vecs = nxt
        acc = vecs[0] + bias
        y = 1.0 / (1.0 + jnp.exp(-acc))
        start = pl.multiple_of(blk * 16, 16)
        out_v[pl.ds(start, 16)] = y
        return carry

    lax.fori_loop(0, BPW // 16, blk_body, 0)

    pltpu.sync_copy(out_v, out_hbm.at[pl.ds(base, BPW)])


@jax.jit
def _gmf_call(uid, iid, wb, user_mat, item_mat):
    mesh = plsc.VectorSubcoreMesh(core_axis_name="c", subcore_axis_name="s")
    run = functools.partial(
        pl.kernel,
        mesh=mesh,
        compiler_params=pltpu.CompilerParams(use_tc_tiling_on_sc=False),
        out_type=jax.ShapeDtypeStruct((B,), jnp.float32),
        scratch_types=[
            pltpu.VMEM((BPW,), jnp.int32),
            pltpu.VMEM((BPW,), jnp.int32),
            pltpu.VMEM((BPW, K), jnp.float32),
            pltpu.VMEM((BPW, K), jnp.float32),
            pltpu.VMEM((K + 16,), jnp.float32),
            pltpu.VMEM((BPW,), jnp.float32),
            pltpu.SemaphoreType.DMA,
        ],
    )(_sc_gmf)
    return run(uid, iid, wb, user_mat, item_mat)


def kernel(uid, iid, user_mat, item_mat, affine_w, affine_b):
    wb = jnp.concatenate([affine_w.reshape(K), affine_b,
                          jnp.zeros((15,), jnp.float32)])
    return _gmf_call(uid, iid, wb, user_mat, item_mat)


# speedup vs baseline: 1.4925x; 1.4925x over previous
import functools

import jax
import jax.numpy as jnp
from jax import lax
from jax.experimental import pallas as pl
from jax.experimental.pallas import tpu as pltpu
from jax.experimental.pallas import tpu_sc as plsc

B = 16384
K = 32
NC = 2
NS = 16
NW = NC * NS
BPW = B // NW
CHUNK = 128
NCHUNK = BPW // CHUNK


def _sc_gmf(uid_hbm, iid_hbm, wb_hbm, user_hbm, item_hbm, out_hbm,
            idx_u, idx_i, u_buf, i_buf, wb_v, out_v, sem_u, sem_i):
    wid = lax.axis_index("s") * NC + lax.axis_index("c")
    base = wid * BPW

    pltpu.sync_copy(uid_hbm.at[pl.ds(base, BPW)], idx_u)
    pltpu.sync_copy(iid_hbm.at[pl.ds(base, BPW)], idx_i)
    pltpu.sync_copy(wb_hbm, wb_v)

    iota16 = lax.iota(jnp.int32, 16)
    w_lo = wb_v[pl.ds(0, 16)]
    w_hi = wb_v[pl.ds(16, 16)]
    b_vec = wb_v[pl.ds(K, 16)]
    bias = jnp.zeros((16,), jnp.float32) + b_vec[0]
    perm_idx = [iota16 ^ s for s in (1, 2, 4, 8)]

    def _xor_perm(v, level):
        return v.at[perm_idx[level]].get(mode="promise_in_bounds",
                                         unique_indices=True)

    def chunk_body(c, carry):
        coff = pl.multiple_of(c * CHUNK, CHUNK)

        def fire_body(g, fcarry):
            goff = pl.multiple_of(coff + g * 16, 16)
            uvec = idx_u[pl.ds(goff, 16)]
            ivec = idx_i[pl.ds(goff, 16)]
            for j in range(16):
                dst = pl.ds(g * 16 + j, 1)
                pltpu.async_copy(user_hbm.at[pl.ds(uvec[j], 1), :],
                                 u_buf.at[dst, :], sem_u)
                pltpu.async_copy(item_hbm.at[pl.ds(ivec[j], 1), :],
                                 i_buf.at[dst, :], sem_i)
            return fcarry

        lax.fori_loop(0, CHUNK // 16, fire_body, 0)

        pltpu.make_async_copy(user_hbm.at[pl.ds(0, CHUNK), :], u_buf,
                              sem_u).wait()
        pltpu.make_async_copy(item_hbm.at[pl.ds(0, CHUNK), :], i_buf,
                              sem_i).wait()

        def blk_body(blk, bcarry):
            vecs = []
            for j in range(16):
                r = blk * 16 + j
                u0 = u_buf[r, pl.ds(0, 16)]
                u1 = u_buf[r, pl.ds(16, 16)]
                i0 = i_buf[r, pl.ds(0, 16)]
                i1 = i_buf[r, pl.ds(16, 16)]
                vecs.append(u0 * i0 * w_lo + u1 * i1 * w_hi)
            for level, s in enumerate((1, 2, 4, 8)):
                lane_bit = (iota16 & s) == 0
                nxt = []
                for j in range(0, len(vecs), 2):
                    a = vecs[j] + _xor_perm(vecs[j], level)
                    b = vecs[j + 1] + _xor_perm(vecs[j + 1], level)
                    nxt.append(jnp.where(lane_bit, a, b))
                vecs = nxt
            acc = vecs[0] + bias
            y = 1.0 / (1.0 + jnp.exp(-acc))
            start = pl.multiple_of(coff + blk * 16, 16)
            out_v[pl.ds(start, 16)] = y
            return bcarry

        lax.fori_loop(0, CHUNK // 16, blk_body, 0)
        return carry

    lax.fori_loop(0, NCHUNK, chunk_body, 0)

    pltpu.sync_copy(out_v, out_hbm.at[pl.ds(base, BPW)])


@jax.jit
def _gmf_call(uid, iid, wb, user_mat, item_mat):
    mesh = plsc.VectorSubcoreMesh(core_axis_name="c", subcore_axis_name="s")
    run = functools.partial(
        pl.kernel,
        mesh=mesh,
        out_type=jax.ShapeDtypeStruct((B,), jnp.float32),
        scratch_types=[
            pltpu.VMEM((BPW,), jnp.int32),
            pltpu.VMEM((BPW,), jnp.int32),
            pltpu.VMEM((CHUNK, K), jnp.float32),
            pltpu.VMEM((CHUNK, K), jnp.float32),
            pltpu.VMEM((K + 16,), jnp.float32),
            pltpu.VMEM((BPW,), jnp.float32),
            pltpu.SemaphoreType.DMA,
            pltpu.SemaphoreType.DMA,
        ],
    )(_sc_gmf)
    return run(uid, iid, wb, user_mat, item_mat)


def kernel(uid, iid, user_mat, item_mat, affine_w, affine_b):
    wb = jnp.concatenate([affine_w.reshape(K), affine_b,
                          jnp.zeros((15,), jnp.float32)])
    return _gmf_call(uid, iid, wb, user_mat, item_mat)
